# baseline (speedup 1.0000x reference)
import functools

import jax
import jax.numpy as jnp
from jax import lax
from jax.experimental import pallas as pl
from jax.experimental.pallas import tpu as pltpu
from jax.experimental.pallas import tpu_sc as plsc

NUM_SAMPLES = 32
C_FRAMES = 3
T = 300
H = 224
W = 224
NC, NS = 2, 16
NW = NC * NS
TC_CLIPS = 2
SC_CLIPS = C_FRAMES - TC_CLIPS


def _sc_part(x):
    mesh = plsc.VectorSubcoreMesh(core_axis_name="c", subcore_axis_name="s")

    @functools.partial(
        pl.kernel,
        mesh=mesh,
        out_type=jax.ShapeDtypeStruct((SC_CLIPS, NUM_SAMPLES, H, W), jnp.float32),
        scratch_types=[pltpu.VMEM((1, 1, H, W), jnp.float32),
                       pltpu.SemaphoreType.DMA],
    )
    def k(x_hbm, out_hbm, buf, sem):
        wid = lax.axis_index("s") * NC + lax.axis_index("c")
        j = wid
        t = (299 * j) // 31
        pltpu.async_copy(
            x_hbm.at[pl.ds(TC_CLIPS, 1), pl.ds(t, 1)], buf, sem
        ).wait()
        pltpu.async_copy(buf, out_hbm.at[pl.ds(0, 1), pl.ds(j, 1)], sem).wait()

    return k(x)


def _tc_part(x):
    def body(x_ref, o_ref):
        o_ref[...] = x_ref[...]

    nf = TC_CLIPS * NUM_SAMPLES
    return pl.pallas_call(
        body,
        grid=(nf,),
        in_specs=[pl.BlockSpec(
            (1, 1, H, W),
            lambda i: (i // NUM_SAMPLES, (299 * (i % NUM_SAMPLES)) // 31, 0, 0),
        )],
        out_specs=pl.BlockSpec(
            (1, 1, H, W),
            lambda i: (i // NUM_SAMPLES, i % NUM_SAMPLES, 0, 0),
        ),
        out_shape=jax.ShapeDtypeStruct((TC_CLIPS, NUM_SAMPLES, H, W), jnp.float32),
    )(x)


def kernel(x):
    tc = _tc_part(x)
    sc = _sc_part(x)
    return jnp.concatenate([tc, sc], axis=0)


# speedup vs baseline: 2.0919x; 2.0919x over previous
import functools

import jax
import jax.numpy as jnp
from jax import lax
from jax.experimental import pallas as pl
from jax.experimental.pallas import tpu as pltpu
from jax.experimental.pallas import tpu_sc as plsc

NUM_SAMPLES = 32
C_FRAMES = 3
T = 300
H = 224
W = 224
NC, NS = 2, 16
NW = NC * NS
NFRAMES = C_FRAMES * NUM_SAMPLES
SPLIT = 2
HP = H // SPLIT
K = NFRAMES * SPLIT // NW
NBUF = 4
D = 2


def _sc_subsample(x):
    mesh = plsc.VectorSubcoreMesh(core_axis_name="c", subcore_axis_name="s")

    @functools.partial(
        pl.kernel,
        mesh=mesh,
        out_type=jax.ShapeDtypeStruct((C_FRAMES, NUM_SAMPLES, H, W), jnp.float32),
        scratch_types=[pltpu.VMEM((NBUF, 1, 1, HP, W), jnp.float32),
                       pltpu.SemaphoreType.DMA,
                       pltpu.SemaphoreType.DMA],
    )
    def k(x_hbm, out_hbm, bufs, rsem, wsem):
        wid = lax.axis_index("s") * NC + lax.axis_index("c")

        def piece_loc(i):
            p = wid * K + i
            f = p // SPLIT
            h0 = (p % SPLIT) * HP
            j = f % NUM_SAMPLES
            c = f // NUM_SAMPLES
            t = (299 * j) // 31
            return c, j, t, h0

        reads = [None] * K
        writes = [None] * K
        for i in range(K + D):
            if i < K:
                b = i % NBUF
                if i >= NBUF:
                    writes[i - NBUF].wait()
                c, j, t, h0 = piece_loc(i)
                reads[i] = pltpu.async_copy(
                    x_hbm.at[pl.ds(c, 1), pl.ds(t, 1), pl.ds(h0, HP)],
                    bufs.at[b],
                    rsem,
                )
            wi = i - D
            if wi >= 0:
                reads[wi].wait()
                pc, pj, _, ph0 = piece_loc(wi)
                writes[wi] = pltpu.async_copy(
                    bufs.at[wi % NBUF],
                    out_hbm.at[pl.ds(pc, 1), pl.ds(pj, 1), pl.ds(ph0, HP)],
                    wsem,
                )
        for i in range(max(0, K - NBUF), K):
            writes[i].wait()

    return k(x)


def kernel(x):
    return _sc_subsample(x)


# speedup vs baseline: 2.1012x; 1.0045x over previous
import functools

import jax
import jax.numpy as jnp
from jax import lax
from jax.experimental import pallas as pl
from jax.experimental.pallas import tpu as pltpu
from jax.experimental.pallas import tpu_sc as plsc

NUM_SAMPLES = 32
C_FRAMES = 3
T = 300
H = 224
W = 224
NC, NS = 2, 16
NW = NC * NS
NFRAMES = C_FRAMES * NUM_SAMPLES
K = NFRAMES // NW
NBUF = 2


def _sc_subsample(x):
    mesh = plsc.VectorSubcoreMesh(core_axis_name="c", subcore_axis_name="s")

    @functools.partial(
        pl.kernel,
        mesh=mesh,
        out_type=jax.ShapeDtypeStruct((C_FRAMES, NUM_SAMPLES, H, W), jnp.float32),
        scratch_types=[pltpu.VMEM((NBUF, 1, 1, H, W), jnp.float32),
                       pltpu.SemaphoreType.DMA,
                       pltpu.SemaphoreType.DMA],
    )
    def k(x_hbm, out_hbm, bufs, rsem, wsem):
        wid = lax.axis_index("s") * NC + lax.axis_index("c")

        def frame_loc(i):
            f = wid * K + i
            j = f % NUM_SAMPLES
            c = f // NUM_SAMPLES
            t = (299 * j) // 31
            return c, j, t

        reads = [None] * K
        writes = [None] * K
        for i in range(K):
            b = i % NBUF
            if i >= NBUF:
                writes[i - NBUF].wait()
            c, j, t = frame_loc(i)
            reads[i] = pltpu.async_copy(
                x_hbm.at[pl.ds(c, 1), pl.ds(t, 1)], bufs.at[b], rsem
            )
            if i >= 1:
                reads[i - 1].wait()
                pc, pj, _ = frame_loc(i - 1)
                writes[i - 1] = pltpu.async_copy(
                    bufs.at[(i - 1) % NBUF],
                    out_hbm.at[pl.ds(pc, 1), pl.ds(pj, 1)],
                    wsem,
                )
        reads[K - 1].wait()
        pc, pj, _ = frame_loc(K - 1)
        writes[K - 1] = pltpu.async_copy(
            bufs.at[(K - 1) % NBUF], out_hbm.at[pl.ds(pc, 1), pl.ds(pj, 1)], wsem
        )
        for i in range(max(0, K - NBUF), K):
            writes[i].wait()

    return k(x)


def kernel(x):
    return _sc_subsample(x)
